# baseline (speedup 1.0000x reference)
import functools

import jax
import jax.numpy as jnp
from jax import lax
from jax.experimental import pallas as pl
from jax.experimental.pallas import tpu as pltpu
from jax.experimental.pallas import tpu_sc as plsc

N_NODES = 50000
N_EDGES = 800000
N_GRAPHS = 512
VOCAB = 100000
EMB = 64
HID = 128

NC, NS = 2, 16
NW = NC * NS
NPAD = 50176
EBLK = 2048
NBLKE = 26
EPT = NBLKE * EBLK
EPAD = EPT * NS
BLK = 512
NBLK = NPAD // BLK

_mesh = plsc.VectorSubcoreMesh(core_axis_name="c", subcore_axis_name="s",
                               num_cores=NC, num_subcores=NS)


GC = 112
RPW = NPAD // NW


@functools.partial(
    pl.kernel,
    out_type=jax.ShapeDtypeStruct((NPAD, HID), jnp.float32),
    mesh=_mesh,
    scratch_types=[
        pltpu.VMEM((RPW,), jnp.int32),
        pltpu.VMEM((GC, HID), jnp.float32),
        pltpu.VMEM((GC, HID), jnp.float32),
        pltpu.SemaphoreType.DMA,
    ],
)
def _emb_kernel(x_hbm, table_hbm, h0_hbm, idx_v, rows0, rows1, sem_g):
    cid = lax.axis_index("c")
    sid = lax.axis_index("s")
    wid = sid * NC + cid
    base = wid * RPW
    rowss = (rows0, rows1)
    pltpu.sync_copy(x_hbm.at[pl.ds(pl.multiple_of(base, RPW), RPW)], idx_v)
    pltpu.async_copy(table_hbm.at[idx_v.at[pl.ds(0, GC)]], rows0, sem_g)
    for j in range(RPW // GC):
        b = j % 2
        if j + 1 < RPW // GC:
            pltpu.async_copy(table_hbm.at[idx_v.at[pl.ds((j + 1) * GC, GC)]],
                             rowss[1 - b], sem_g)
        pltpu.make_async_copy(table_hbm, rowss[b], sem_g).wait()
        r0 = pl.multiple_of(base + j * GC, 8)
        pltpu.sync_copy(rowss[b], h0_hbm.at[pl.ds(r0, GC)])


def _make_agg(chunk_nodes, chunks_per_core):
    rpt = chunk_nodes // NS
    trash = chunk_nodes
    accr = chunk_nodes + 8

    @functools.partial(
        pl.kernel,
        out_type=jax.ShapeDtypeStruct((NPAD, HID), jnp.float32),
        mesh=_mesh,
        scratch_types=[
            pltpu.VMEM((EBLK,), jnp.int32),
            pltpu.VMEM((EBLK,), jnp.int32),
            pltpu.VMEM((128,), jnp.int32),
            pltpu.VMEM((128, HID), jnp.float32),
            pltpu.VMEM_SHARED((accr, HID), jnp.float32),
        ],
    )
    def _agg(src_hbm, dst_hbm, h_hbm, z_hbm, out_hbm, srcv, dstv, cdstj, rows, acc):
        cid = lax.axis_index("c")
        sid = lax.axis_index("s")
        ebase = sid * EPT
        for ci in range(chunks_per_core):
            chunk_id = cid * chunks_per_core + ci
            lo = chunk_id * chunk_nodes
            pltpu.sync_copy(z_hbm.at[pl.ds(0, rpt)],
                            acc.at[pl.ds(pl.multiple_of(sid * rpt, 8), rpt)])

            @pl.when(sid == 0)
            def _():
                pltpu.sync_copy(z_hbm.at[pl.ds(0, 8)],
                                acc.at[pl.ds(chunk_nodes, 8)])

            plsc.subcore_barrier()

            @pl.loop(0, EPT // EBLK)
            def _blk(b):
                e0 = pl.multiple_of(ebase + b * EBLK, EBLK)
                pltpu.sync_copy(src_hbm.at[pl.ds(e0, EBLK)], srcv)
                pltpu.sync_copy(dst_hbm.at[pl.ds(e0, EBLK)], dstv)

                @pl.loop(0, EBLK // 128)
                def _gth(j):
                    joff = pl.multiple_of(j * 128, 128)
                    for i2 in range(8):
                        d = dstv[pl.ds(joff + i2 * 16, 16)]
                        m = (d >= lo) & (d < lo + chunk_nodes)
                        cdstj[pl.ds(i2 * 16, 16)] = jnp.where(m, d - lo, trash)
                    pltpu.sync_copy(h_hbm.at[srcv.at[pl.ds(joff, 128)]], rows)
                    pltpu.sync_copy(rows, acc.at[cdstj], add=True)

            plsc.subcore_barrier()
            r0 = pl.multiple_of(sid * rpt, 8)
            pltpu.sync_copy(acc.at[pl.ds(r0, rpt)],
                            out_hbm.at[pl.ds(lo + r0, rpt)])
            if ci + 1 < chunks_per_core:
                plsc.subcore_barrier()

    return _agg


_agg_l1 = _make_agg(NPAD // 4, 2)
_agg_l2 = _agg_l1


def _tc1_body(h0_ref, agg_ref, wl_ref, b_ref, wr_ref, out_ref):
    deg = jnp.maximum(agg_ref[:, EMB:EMB + 1], 1.0)
    mean = agg_ref[:, :EMB] / deg
    out_ref[...] = jnp.maximum(
        jnp.dot(mean, wl_ref[...], preferred_element_type=jnp.float32)
        + b_ref[...]
        + jnp.dot(h0_ref[:, :EMB], wr_ref[...],
                  preferred_element_type=jnp.float32),
        0.0)


def _tc2_body(agg2_ref, agg1_ref, h1_ref, batch_ref, wl_ref, b_ref, wr_ref,
              wlin_ref, blin_ref, out_ref, gsum, gcnt):
    i = pl.program_id(0)

    @pl.when(i == 0)
    def _():
        gsum[...] = jnp.zeros_like(gsum)
        gcnt[...] = jnp.zeros_like(gcnt)

    deg = jnp.maximum(agg1_ref[:, EMB:EMB + 1], 1.0)
    mean = agg2_ref[...] / deg
    h2 = jnp.maximum(
        jnp.dot(mean, wl_ref[...], preferred_element_type=jnp.float32)
        + b_ref[...]
        + jnp.dot(h1_ref[...], wr_ref[...], preferred_element_type=jnp.float32),
        0.0)
    bids = batch_ref[0, 0, :]
    gid = lax.broadcasted_iota(jnp.int32, (N_GRAPHS, BLK), 0)
    onehot = (gid == bids[None, :]).astype(jnp.float32)
    gsum[...] = gsum[...] + jnp.dot(onehot, h2,
                                    preferred_element_type=jnp.float32)
    gcnt[...] = gcnt[...] + jnp.sum(onehot, axis=1, keepdims=True)

    @pl.when(i == NBLK - 1)
    def _():
        g = gsum[...] / jnp.maximum(gcnt[...], 1.0)
        out_ref[...] = (jnp.dot(g, wlin_ref[...],
                                preferred_element_type=jnp.float32)
                        + blin_ref[...])


def kernel(x, edge_index, batch, emb_table, W1l, b1, W1r, W2l, b2, W2r,
           Wlin, blin):
    x = x.astype(jnp.int32)
    xp = jnp.pad(x, (0, NPAD - N_NODES))
    src = jnp.pad(edge_index[0].astype(jnp.int32), (0, EPAD - N_EDGES))
    dst = jnp.pad(edge_index[1].astype(jnp.int32), (0, EPAD - N_EDGES),
                  constant_values=NPAD)
    batchp = jnp.pad(batch.astype(jnp.int32), (0, NPAD - N_NODES),
                     constant_values=N_GRAPHS).reshape(NBLK, 1, BLK)
    tpad = jnp.concatenate(
        [emb_table, jnp.ones((VOCAB, 1), jnp.float32),
         jnp.zeros((VOCAB, HID - EMB - 1), jnp.float32)], axis=1)
    z = jnp.zeros((NPAD // 4 // NS, HID), jnp.float32)

    h0 = _emb_kernel(xp, tpad)
    agg1 = _agg_l1(src, dst, h0, z)

    h1 = pl.pallas_call(
        _tc1_body,
        grid=(NBLK,),
        in_specs=[
            pl.BlockSpec((BLK, HID), lambda i: (i, 0)),
            pl.BlockSpec((BLK, HID), lambda i: (i, 0)),
            pl.BlockSpec((EMB, HID), lambda i: (0, 0)),
            pl.BlockSpec((1, HID), lambda i: (0, 0)),
            pl.BlockSpec((EMB, HID), lambda i: (0, 0)),
        ],
        out_specs=pl.BlockSpec((BLK, HID), lambda i: (i, 0)),
        out_shape=jax.ShapeDtypeStruct((NPAD, HID), jnp.float32),
    )(h0, agg1, W1l, b1.reshape(1, HID), W1r)

    agg2 = _agg_l2(src, dst, h1, z)

    out = pl.pallas_call(
        _tc2_body,
        grid=(NBLK,),
        in_specs=[
            pl.BlockSpec((BLK, HID), lambda i: (i, 0)),
            pl.BlockSpec((BLK, HID), lambda i: (i, 0)),
            pl.BlockSpec((BLK, HID), lambda i: (i, 0)),
            pl.BlockSpec((1, 1, BLK), lambda i: (i, 0, 0)),
            pl.BlockSpec((HID, HID), lambda i: (0, 0)),
            pl.BlockSpec((1, HID), lambda i: (0, 0)),
            pl.BlockSpec((HID, HID), lambda i: (0, 0)),
            pl.BlockSpec((HID, 2), lambda i: (0, 0)),
            pl.BlockSpec((1, 2), lambda i: (0, 0)),
        ],
        out_specs=pl.BlockSpec((N_GRAPHS, 2), lambda i: (0, 0)),
        out_shape=jax.ShapeDtypeStruct((N_GRAPHS, 2), jnp.float32),
        scratch_shapes=[
            pltpu.VMEM((N_GRAPHS, HID), jnp.float32),
            pltpu.VMEM((N_GRAPHS, 1), jnp.float32),
        ],
    )(agg2, agg1, h1, batchp, W2l, b2.reshape(1, HID), W2r, Wlin,
      blin.reshape(1, 2))
    return out


# speedup vs baseline: 3.4870x; 3.4870x over previous
import functools

import jax
import jax.numpy as jnp
from jax import lax
from jax.experimental import pallas as pl
from jax.experimental.pallas import tpu as pltpu
from jax.experimental.pallas import tpu_sc as plsc

N_NODES = 50000
N_EDGES = 800000
N_GRAPHS = 512
VOCAB = 100000
EMB = 64
HID = 128

NC, NS = 2, 16
NW = NC * NS
NPAD = 50176
NCHUNK = 4
CHN = NPAD // NCHUNK
EPAD = 800768
EPS = EPAD // NS
EBLK = EPS // 128
BLK = 512
NBLK = NPAD // BLK
RPT = CHN // NS

_mesh = plsc.VectorSubcoreMesh(core_axis_name="c", subcore_axis_name="s",
                               num_cores=NC, num_subcores=NS)


GC = 112
RPW = NPAD // NW


@functools.partial(
    pl.kernel,
    out_type=jax.ShapeDtypeStruct((NPAD, HID), jnp.float32),
    mesh=_mesh,
    scratch_types=[
        pltpu.VMEM((RPW,), jnp.int32),
        pltpu.VMEM((GC, HID), jnp.float32),
    ],
)
def _emb_kernel(x_hbm, table_hbm, h0_hbm, idx_v, rows0):
    cid = lax.axis_index("c")
    sid = lax.axis_index("s")
    wid = sid * NC + cid
    base = wid * RPW
    pltpu.sync_copy(x_hbm.at[pl.ds(pl.multiple_of(base, RPW), RPW)], idx_v)
    for j in range(RPW // GC):
        pltpu.sync_copy(table_hbm.at[idx_v.at[pl.ds(j * GC, GC)]], rows0)
        r0 = pl.multiple_of(base + j * GC, 8)
        pltpu.sync_copy(rows0, h0_hbm.at[pl.ds(r0, GC)])


ACCR = CHN + 8


@functools.partial(
    pl.kernel,
    out_type=jax.ShapeDtypeStruct((NPAD, HID), jnp.float32),
    mesh=_mesh,
    scratch_types=[
        pltpu.VMEM((128,), jnp.int32),
        pltpu.VMEM((128,), jnp.int32),
        pltpu.VMEM((128,), jnp.int32),
        pltpu.VMEM((128, HID), jnp.float32),
        pltpu.VMEM_SHARED((ACCR, HID), jnp.float32),
    ],
)
def _agg_kernel(src_hbm, dst_hbm, h_hbm, z_hbm, out_hbm,
                srcv, dstv, ldst, rows, acc):
    cid = lax.axis_index("c")
    sid = lax.axis_index("s")
    for ci in range(NCHUNK // NC):
        b = cid * (NCHUNK // NC) + ci
        lo = b * CHN
        pltpu.sync_copy(z_hbm.at[pl.ds(0, RPT)],
                        acc.at[pl.ds(pl.multiple_of(sid * RPT, 8), RPT)])

        @pl.when(sid == 0)
        def _():
            pltpu.sync_copy(z_hbm.at[pl.ds(0, 8)], acc.at[pl.ds(CHN, 8)])

        plsc.subcore_barrier()

        def blk(j, _):
            e0 = sid * EPS + j * 128
            pltpu.sync_copy(src_hbm.at[pl.ds(e0, 128)], srcv)
            pltpu.sync_copy(dst_hbm.at[pl.ds(e0, 128)], dstv)
            for g in range(8):
                d16 = dstv[pl.ds(g * 16, 16)]
                m = (d16 >= lo) & (d16 < lo + CHN)
                ldst[pl.ds(g * 16, 16)] = jnp.where(m, d16 - lo, CHN)
            pltpu.sync_copy(h_hbm.at[srcv], rows)
            pltpu.sync_copy(rows, acc.at[ldst], add=True)
            return 0

        lax.fori_loop(0, EBLK, blk, 0)

        plsc.subcore_barrier()
        r0 = pl.multiple_of(sid * RPT, 8)
        pltpu.sync_copy(acc.at[pl.ds(r0, RPT)],
                        out_hbm.at[pl.ds(lo + r0, RPT)])
        if ci + 1 < NCHUNK // NC:
            plsc.subcore_barrier()


def _tc1_body(h0_ref, agg_ref, wl_ref, b_ref, wr_ref, out_ref):
    deg = jnp.maximum(agg_ref[:, EMB:EMB + 1], 1.0)
    mean = agg_ref[:, :EMB] / deg
    out_ref[...] = jnp.maximum(
        jnp.dot(mean, wl_ref[...], preferred_element_type=jnp.float32)
        + b_ref[...]
        + jnp.dot(h0_ref[:, :EMB], wr_ref[...],
                  preferred_element_type=jnp.float32),
        0.0)


def _tc2_body(agg2_ref, agg1_ref, h1_ref, batch_ref, wl_ref, b_ref, wr_ref,
              wlin_ref, blin_ref, out_ref, gsum, gcnt):
    i = pl.program_id(0)

    @pl.when(i == 0)
    def _():
        gsum[...] = jnp.zeros_like(gsum)
        gcnt[...] = jnp.zeros_like(gcnt)

    deg = jnp.maximum(agg1_ref[:, EMB:EMB + 1], 1.0)
    mean = agg2_ref[...] / deg
    h2 = jnp.maximum(
        jnp.dot(mean, wl_ref[...], preferred_element_type=jnp.float32)
        + b_ref[...]
        + jnp.dot(h1_ref[...], wr_ref[...], preferred_element_type=jnp.float32),
        0.0)
    bids = batch_ref[0, 0, :]
    gid = lax.broadcasted_iota(jnp.int32, (N_GRAPHS, BLK), 0)
    onehot = (gid == bids[None, :]).astype(jnp.float32)
    gsum[...] = gsum[...] + jnp.dot(onehot, h2,
                                    preferred_element_type=jnp.float32)
    gcnt[...] = gcnt[...] + jnp.sum(onehot, axis=1, keepdims=True)

    @pl.when(i == NBLK - 1)
    def _():
        g = gsum[...] / jnp.maximum(gcnt[...], 1.0)
        out_ref[...] = (jnp.dot(g, wlin_ref[...],
                                preferred_element_type=jnp.float32)
                        + blin_ref[...])


def kernel(x, edge_index, batch, emb_table, W1l, b1, W1r, W2l, b2, W2r,
           Wlin, blin):
    x = x.astype(jnp.int32)
    xp = jnp.pad(x, (0, NPAD - N_NODES))
    src = jnp.pad(edge_index[0].astype(jnp.int32), (0, EPAD - N_EDGES))
    dst = jnp.pad(edge_index[1].astype(jnp.int32), (0, EPAD - N_EDGES),
                  constant_values=NPAD)
    batchp = jnp.pad(batch.astype(jnp.int32), (0, NPAD - N_NODES),
                     constant_values=N_GRAPHS).reshape(NBLK, 1, BLK)
    tpad = jnp.concatenate(
        [emb_table, jnp.ones((VOCAB, 1), jnp.float32),
         jnp.zeros((VOCAB, HID - EMB - 1), jnp.float32)], axis=1)
    z = jnp.zeros((RPT, HID), jnp.float32)

    h0 = _emb_kernel(xp, tpad)
    agg1 = _agg_kernel(src, dst, h0, z)

    h1 = pl.pallas_call(
        _tc1_body,
        grid=(NBLK,),
        in_specs=[
            pl.BlockSpec((BLK, HID), lambda i: (i, 0)),
            pl.BlockSpec((BLK, HID), lambda i: (i, 0)),
            pl.BlockSpec((EMB, HID), lambda i: (0, 0)),
            pl.BlockSpec((1, HID), lambda i: (0, 0)),
            pl.BlockSpec((EMB, HID), lambda i: (0, 0)),
        ],
        out_specs=pl.BlockSpec((BLK, HID), lambda i: (i, 0)),
        out_shape=jax.ShapeDtypeStruct((NPAD, HID), jnp.float32),
    )(h0, agg1, W1l, b1.reshape(1, HID), W1r)

    agg2 = _agg_kernel(src, dst, h1, z)

    out = pl.pallas_call(
        _tc2_body,
        grid=(NBLK,),
        in_specs=[
            pl.BlockSpec((BLK, HID), lambda i: (i, 0)),
            pl.BlockSpec((BLK, HID), lambda i: (i, 0)),
            pl.BlockSpec((BLK, HID), lambda i: (i, 0)),
            pl.BlockSpec((1, 1, BLK), lambda i: (i, 0, 0)),
            pl.BlockSpec((HID, HID), lambda i: (0, 0)),
            pl.BlockSpec((1, HID), lambda i: (0, 0)),
            pl.BlockSpec((HID, HID), lambda i: (0, 0)),
            pl.BlockSpec((HID, 2), lambda i: (0, 0)),
            pl.BlockSpec((1, 2), lambda i: (0, 0)),
        ],
        out_specs=pl.BlockSpec((N_GRAPHS, 2), lambda i: (0, 0)),
        out_shape=jax.ShapeDtypeStruct((N_GRAPHS, 2), jnp.float32),
        scratch_shapes=[
            pltpu.VMEM((N_GRAPHS, HID), jnp.float32),
            pltpu.VMEM((N_GRAPHS, 1), jnp.float32),
        ],
    )(agg2, agg1, h1, batchp, W2l, b2.reshape(1, HID), W2r, Wlin,
      blin.reshape(1, 2))
    return out
